# baseline (speedup 1.0000x reference)
import functools

import jax
import jax.numpy as jnp
from jax import lax
from jax.experimental import pallas as pl
from jax.experimental.pallas import tpu as pltpu
from jax.experimental.pallas import tpu_sc as plsc

_B = 16384
_D = 16
_N = 1000000
_NC = 2
_NS = 16
_NW = _NC * _NS
_BPW = _B // _NW
_K = 16
_TMAX = _N // 128 - 1
_SIDE0 = _TMAX * 128
_SIDEW = 256


def _body(user_hbm, item_hbm, utab_hbm, itab_hbm, uside_hbm, iside_hbm,
          out_hbm, idx_u, idx_i, ring_u, ring_i, side_u, side_i, dots, sem):
    wid = lax.axis_index("s") * _NC + lax.axis_index("c")
    base = wid * _BPW

    pltpu.sync_copy(user_hbm.at[pl.ds(base, _BPW)], idx_u)
    pltpu.sync_copy(item_hbm.at[pl.ds(base, _BPW)], idx_i)
    pltpu.sync_copy(uside_hbm, side_u)
    pltpu.sync_copy(iside_hbm, side_i)

    lane = lax.iota(jnp.int32, _D)
    g_idx = lane // 8
    s_idx = lane % 8

    def batch(b, carry):
        i0 = b * _K
        rv_u = idx_u[pl.ds(i0, _K)]
        rv_i = idx_i[pl.ds(i0, _K)]
        cps = []
        for j in range(_K):
            r_u = jnp.sum(jnp.where(lane == j, rv_u, 0), axis=0)
            t_u = jnp.minimum(r_u // 128, _TMAX)
            cps.append(pltpu.async_copy(
                utab_hbm.at[:, :, pl.ds(t_u * 128, 128)], ring_u.at[j], sem))
            r_i = jnp.sum(jnp.where(lane == j, rv_i, 0), axis=0)
            t_i = jnp.minimum(r_i // 128, _TMAX)
            cps.append(pltpu.async_copy(
                itab_hbm.at[:, :, pl.ds(t_i * 128, 128)], ring_i.at[j], sem))
        for cp in cps:
            cp.wait()
        acc = jnp.zeros((_D,), jnp.float32)
        for j in range(_K):
            r_u = jnp.sum(jnp.where(lane == j, rv_u, 0), axis=0)
            t_u = jnp.minimum(r_u // 128, _TMAX)
            col_u = jnp.zeros((_D,), jnp.int32) + jnp.minimum(
                r_u - t_u * 128, 127)
            cs_u = jnp.zeros((_D,), jnp.int32) + jnp.maximum(r_u - _SIDE0, 0)
            v_main = plsc.load_gather(ring_u.at[j], [g_idx, s_idx, col_u])
            v_side = plsc.load_gather(side_u, [g_idx, s_idx, cs_u])
            vu = jnp.where(r_u < _SIDE0, v_main, v_side)

            r_i = jnp.sum(jnp.where(lane == j, rv_i, 0), axis=0)
            t_i = jnp.minimum(r_i // 128, _TMAX)
            col_i = jnp.zeros((_D,), jnp.int32) + jnp.minimum(
                r_i - t_i * 128, 127)
            cs_i = jnp.zeros((_D,), jnp.int32) + jnp.maximum(r_i - _SIDE0, 0)
            w_main = plsc.load_gather(ring_i.at[j], [g_idx, s_idx, col_i])
            w_side = plsc.load_gather(side_i, [g_idx, s_idx, cs_i])
            vi = jnp.where(r_i < _SIDE0, w_main, w_side)

            acc = jnp.where(lane == j, jnp.sum(vu * vi, axis=0), acc)
        dots[pl.ds(i0, _K)] = acc
        return carry

    lax.fori_loop(0, _BPW // _K, batch, 0)

    def sig(k, carry):
        v = dots[pl.ds(k * _D, _D)]
        dots[pl.ds(k * _D, _D)] = 1.0 / (1.0 + jnp.exp(-v))
        return carry

    lax.fori_loop(0, _BPW // _D, sig, 0)

    pltpu.sync_copy(dots, out_hbm.at[pl.ds(base, _BPW)])


def kernel(user, item, user_table, item_table):
    utab3 = user_table.T.reshape(2, 8, _N)
    itab3 = item_table.T.reshape(2, 8, _N)
    npad = _SIDEW - (_N - _SIDE0)
    uside = jnp.pad(user_table[_SIDE0:].T, ((0, 0), (0, npad))
                    ).reshape(2, 8, _SIDEW)
    iside = jnp.pad(item_table[_SIDE0:].T, ((0, 0), (0, npad))
                    ).reshape(2, 8, _SIDEW)
    mesh = plsc.VectorSubcoreMesh(core_axis_name="c", subcore_axis_name="s")
    f = functools.partial(
        pl.kernel,
        out_type=jax.ShapeDtypeStruct((_B,), jnp.float32),
        mesh=mesh,
        scratch_types=[
            pltpu.VMEM((_BPW,), jnp.int32),
            pltpu.VMEM((_BPW,), jnp.int32),
            pltpu.VMEM((_K, 2, 8, 128), jnp.float32),
            pltpu.VMEM((_K, 2, 8, 128), jnp.float32),
            pltpu.VMEM((2, 8, _SIDEW), jnp.float32),
            pltpu.VMEM((2, 8, _SIDEW), jnp.float32),
            pltpu.VMEM((_BPW,), jnp.float32),
            pltpu.SemaphoreType.DMA,
        ],
        compiler_params=pltpu.CompilerParams(
            needs_layout_passes=False, use_tc_tiling_on_sc=True),
    )(_body)
    return f(user.astype(jnp.int32), item.astype(jnp.int32),
             utab3, itab3, uside, iside)


# speedup vs baseline: 1.0415x; 1.0415x over previous
import functools

import jax
import jax.numpy as jnp
from jax import lax
from jax.experimental import pallas as pl
from jax.experimental.pallas import tpu as pltpu
from jax.experimental.pallas import tpu_sc as plsc

_B = 16384
_D = 16
_N = 1000000
_NC = 2
_NS = 16
_NW = _NC * _NS
_BPW = _B // _NW
_K = 16
_TMAX = _N // 128 - 1
_SIDE0 = _TMAX * 128
_SIDEW = 256


def _body(user_hbm, item_hbm, utab_hbm, itab_hbm, uside_hbm, iside_hbm,
          out_hbm, idx_u, idx_i, ring_u, ring_i, side_u, side_i, dots,
          sem, sem2):
    wid = lax.axis_index("s") * _NC + lax.axis_index("c")
    base = wid * _BPW

    pltpu.sync_copy(user_hbm.at[pl.ds(base, _BPW)], idx_u)
    pltpu.sync_copy(item_hbm.at[pl.ds(base, _BPW)], idx_i)
    pltpu.sync_copy(uside_hbm, side_u)
    pltpu.sync_copy(iside_hbm, side_i)

    lane = lax.iota(jnp.int32, _D)
    g_idx = lane // 8
    s_idx = lane % 8

    def fire(rv_u, rv_i, half, lanes, sem_p):
        for j in lanes:
            r_u = jnp.sum(jnp.where(lane == j, rv_u, 0), axis=0)
            t_u = jnp.minimum(r_u // 128, _TMAX)
            pltpu.async_copy(utab_hbm.at[:, :, pl.ds(t_u * 128, 128)],
                             ring_u.at[half * 8 + (j % 8)], sem_p)
            r_i = jnp.sum(jnp.where(lane == j, rv_i, 0), axis=0)
            t_i = jnp.minimum(r_i // 128, _TMAX)
            pltpu.async_copy(itab_hbm.at[:, :, pl.ds(t_i * 128, 128)],
                             ring_i.at[half * 8 + (j % 8)], sem_p)

    def drain(half, sem_p):
        for j in range(8):
            pltpu.make_async_copy(utab_hbm.at[:, :, pl.ds(0, 128)],
                                  ring_u.at[half * 8 + j], sem_p).wait()
            pltpu.make_async_copy(itab_hbm.at[:, :, pl.ds(0, 128)],
                                  ring_i.at[half * 8 + j], sem_p).wait()

    def extract(rv_u, rv_i, half, lanes, acc):
        for j in lanes:
            r_u = jnp.sum(jnp.where(lane == j, rv_u, 0), axis=0)
            t_u = jnp.minimum(r_u // 128, _TMAX)
            col_u = jnp.zeros((_D,), jnp.int32) + jnp.minimum(
                r_u - t_u * 128, 127)
            cs_u = jnp.zeros((_D,), jnp.int32) + jnp.maximum(r_u - _SIDE0, 0)
            v_main = plsc.load_gather(ring_u.at[half * 8 + (j % 8)],
                                      [g_idx, s_idx, col_u])
            v_side = plsc.load_gather(side_u, [g_idx, s_idx, cs_u])
            vu = jnp.where(r_u < _SIDE0, v_main, v_side)

            r_i = jnp.sum(jnp.where(lane == j, rv_i, 0), axis=0)
            t_i = jnp.minimum(r_i // 128, _TMAX)
            col_i = jnp.zeros((_D,), jnp.int32) + jnp.minimum(
                r_i - t_i * 128, 127)
            cs_i = jnp.zeros((_D,), jnp.int32) + jnp.maximum(r_i - _SIDE0, 0)
            w_main = plsc.load_gather(ring_i.at[half * 8 + (j % 8)],
                                      [g_idx, s_idx, col_i])
            w_side = plsc.load_gather(side_i, [g_idx, s_idx, cs_i])
            vi = jnp.where(r_i < _SIDE0, w_main, w_side)

            acc = jnp.where(lane == j, jnp.sum(vu * vi, axis=0), acc)
        return acc

    npair = _BPW // _K
    rv_u0 = idx_u[pl.ds(0, _K)]
    rv_i0 = idx_i[pl.ds(0, _K)]
    fire(rv_u0, rv_i0, 0, range(8), sem)

    def pair(p, carry):
        i0 = p * _K
        rv_u = idx_u[pl.ds(i0, _K)]
        rv_i = idx_i[pl.ds(i0, _K)]
        fire(rv_u, rv_i, 1, range(8, 16), sem2)
        drain(0, sem)
        acc = extract(rv_u, rv_i, 0, range(8), jnp.zeros((_D,), jnp.float32))

        @pl.when(p < npair - 1)
        def _():
            i1 = jnp.minimum(p + 1, npair - 1) * _K
            nv_u = idx_u[pl.ds(i1, _K)]
            nv_i = idx_i[pl.ds(i1, _K)]
            fire(nv_u, nv_i, 0, range(8), sem)

        drain(1, sem2)
        acc = extract(rv_u, rv_i, 1, range(8, 16), acc)
        dots[pl.ds(i0, _K)] = acc
        return carry

    lax.fori_loop(0, npair, pair, 0)

    def sig(k, carry):
        v = dots[pl.ds(k * _D, _D)]
        dots[pl.ds(k * _D, _D)] = 1.0 / (1.0 + jnp.exp(-v))
        return carry

    lax.fori_loop(0, _BPW // _D, sig, 0)

    pltpu.sync_copy(dots, out_hbm.at[pl.ds(base, _BPW)])


def kernel(user, item, user_table, item_table):
    utab3 = user_table.T.reshape(2, 8, _N)
    itab3 = item_table.T.reshape(2, 8, _N)
    npad = _SIDEW - (_N - _SIDE0)
    uside = jnp.pad(user_table[_SIDE0:].T, ((0, 0), (0, npad))
                    ).reshape(2, 8, _SIDEW)
    iside = jnp.pad(item_table[_SIDE0:].T, ((0, 0), (0, npad))
                    ).reshape(2, 8, _SIDEW)
    mesh = plsc.VectorSubcoreMesh(core_axis_name="c", subcore_axis_name="s")
    f = functools.partial(
        pl.kernel,
        out_type=jax.ShapeDtypeStruct((_B,), jnp.float32),
        mesh=mesh,
        scratch_types=[
            pltpu.VMEM((_BPW,), jnp.int32),
            pltpu.VMEM((_BPW,), jnp.int32),
            pltpu.VMEM((_K, 2, 8, 128), jnp.float32),
            pltpu.VMEM((_K, 2, 8, 128), jnp.float32),
            pltpu.VMEM((2, 8, _SIDEW), jnp.float32),
            pltpu.VMEM((2, 8, _SIDEW), jnp.float32),
            pltpu.VMEM((_BPW,), jnp.float32),
            pltpu.SemaphoreType.DMA,
            pltpu.SemaphoreType.DMA,
        ],
        compiler_params=pltpu.CompilerParams(
            needs_layout_passes=False, use_tc_tiling_on_sc=True),
    )(_body)
    return f(user.astype(jnp.int32), item.astype(jnp.int32),
             utab3, itab3, uside, iside)
